# baseline (speedup 1.0000x reference)
import dataclasses
import functools

import jax
import jax.numpy as jnp
from jax import lax
from jax.experimental import pallas as pl
from jax.experimental.pallas import tpu as pltpu
from jax.experimental.pallas import tpu_sc as plsc

N = 10000
D = 128
E = 320000

NSUB = 16
N1 = 10240
NS = N1 // NSUB
CHUNK = 128
R = 2560
RPS = R // NSUB
EPAD = R * CHUNK - E
G = 32
NG = RPS // G


def _sc_degree_body(dst_hbm, deg_hbm, idx_v, deg_local, red_v, cout_v, deg_sh):
    cid = lax.axis_index("c")
    sid = lax.axis_index("s")

    pltpu.sync_copy(dst_hbm.at[cid, pl.ds(sid * RPS, RPS)], idx_v)

    @pl.loop(0, N1 // 16)
    def _(i):
        deg_local[pl.ds(i * 16, 16)] = jnp.zeros((16,), jnp.float32)

    ones16 = jnp.ones((16,), jnp.float32)

    @pl.loop(0, RPS)
    def _(i):
        @pl.loop(0, CHUNK // 16)
        def _(j):
            idx16 = idx_v[i, pl.ds(j * 16, 16)]
            plsc.addupdate_scatter(deg_local, [idx16], ones16)

    pltpu.sync_copy(deg_local, deg_sh.at[sid])
    plsc.subcore_barrier()
    pltpu.sync_copy(deg_sh.at[:, pl.ds(sid * NS, NS)], red_v)

    @pl.loop(0, NS // 16)
    def _(k):
        acc = red_v[0, pl.ds(k * 16, 16)]
        for rsub in range(1, NSUB):
            acc = acc + red_v[rsub, pl.ds(k * 16, 16)]
        cout_v[pl.ds(k * 16, 16)] = acc

    pltpu.sync_copy(cout_v, deg_hbm.at[pl.ds(cid * N1 + sid * NS, NS)])


def _sc_degree(dst_h):
    mesh = plsc.VectorSubcoreMesh(core_axis_name="c", subcore_axis_name="s")
    cp = pltpu.CompilerParams()
    if "needs_layout_passes" in pltpu.CompilerParams.__dataclass_fields__:
        cp = dataclasses.replace(cp, needs_layout_passes=False)
    fn = pl.kernel(
        _sc_degree_body,
        out_type=jax.ShapeDtypeStruct((2 * N1,), jnp.float32),
        mesh=mesh,
        compiler_params=cp,
        scratch_types=[
            pltpu.VMEM((RPS, CHUNK), jnp.int32),
            pltpu.VMEM((N1,), jnp.float32),
            pltpu.VMEM((NSUB, NS), jnp.float32),
            pltpu.VMEM((NS,), jnp.float32),
            pltpu.VMEM_SHARED((NSUB, N1), jnp.float32),
        ],
    )
    return fn(dst_h)


def _sc_scatter_body(src_hbm, dst_hbm, y_hbm, acc_hbm,
                     src_st, dst_st, rows_a, rows_b,
                     acc_sh, sem_a, sem_b, ssem_a, ssem_b):
    cid = lax.axis_index("c")
    sid = lax.axis_index("s")

    pltpu.sync_copy(y_hbm.at[pl.ds(cid * N1 + sid * NS, NS)],
                    acc_sh.at[pl.ds(sid * NS, NS)])
    plsc.subcore_barrier()

    def gath(r, rows_v, gsem):
        pltpu.async_copy(y_hbm.at[src_st.at[r]], rows_v, gsem)

    def wait_gath(r, rows_v, gsem):
        pltpu.make_async_copy(y_hbm.at[src_st.at[r]], rows_v, gsem).wait()

    def scat(r, rows_v, ssem):
        pltpu.async_copy(rows_v, acc_sh.at[dst_st.at[r]], ssem, add=True)

    def wait_scat(r, rows_v, ssem):
        pltpu.make_async_copy(rows_v, acc_sh.at[dst_st.at[r]], ssem).wait()

    @pl.loop(0, NG)
    def _(g):
        base = sid * RPS + g * G
        pltpu.sync_copy(src_hbm.at[cid, pl.ds(base, G)], src_st)
        pltpu.sync_copy(dst_hbm.at[cid, pl.ds(base, G)], dst_st)

        gath(0, rows_a, sem_a)
        gath(1, rows_b, sem_b)

        @pl.loop(0, G // 2 - 1)
        def _(i):
            wait_gath(2 * i, rows_a, sem_a)
            scat(2 * i, rows_a, ssem_a)
            wait_gath(2 * i + 1, rows_b, sem_b)
            scat(2 * i + 1, rows_b, ssem_b)
            wait_scat(2 * i, rows_a, ssem_a)
            gath(2 * i + 2, rows_a, sem_a)
            wait_scat(2 * i + 1, rows_b, ssem_b)
            gath(2 * i + 3, rows_b, sem_b)

        wait_gath(G - 2, rows_a, sem_a)
        scat(G - 2, rows_a, ssem_a)
        wait_gath(G - 1, rows_b, sem_b)
        scat(G - 1, rows_b, ssem_b)
        wait_scat(G - 2, rows_a, ssem_a)
        wait_scat(G - 1, rows_b, ssem_b)

    plsc.subcore_barrier()
    pltpu.sync_copy(acc_sh.at[pl.ds(sid * NS, NS)],
                    acc_hbm.at[pl.ds(cid * N1 + sid * NS, NS)])


def _sc_scatter(src_h, dst_h, y_flat):
    mesh = plsc.VectorSubcoreMesh(core_axis_name="c", subcore_axis_name="s")
    fn = pl.kernel(
        _sc_scatter_body,
        out_type=jax.ShapeDtypeStruct((2 * N1, D), jnp.float32),
        mesh=mesh,
        scratch_types=[
            pltpu.VMEM((G, CHUNK), jnp.int32),
            pltpu.VMEM((G, CHUNK), jnp.int32),
            pltpu.VMEM((CHUNK, D), jnp.float32),
            pltpu.VMEM((CHUNK, D), jnp.float32),
            pltpu.VMEM_SHARED((N1, D), jnp.float32),
            pltpu.SemaphoreType.DMA,
            pltpu.SemaphoreType.DMA,
            pltpu.SemaphoreType.DMA,
            pltpu.SemaphoreType.DMA,
        ],
    )
    return fn(src_h, dst_h, y_flat)


def _tc_scale_body(h_ref, w0_ref, w1_ref, deg_ref, y_ref):
    dn = (((1,), (0,)), ((), ()))
    h = h_ref[...]
    xw0 = lax.dot_general(h, w0_ref[...], dn,
                          precision=lax.Precision.HIGHEST,
                          preferred_element_type=jnp.float32)
    xw1 = lax.dot_general(h, w1_ref[...], dn,
                          precision=lax.Precision.HIGHEST,
                          preferred_element_type=jnp.float32)
    n0 = lax.rsqrt(deg_ref[0:N, :] + 1.0)
    n1 = lax.rsqrt(deg_ref[N1:N1 + N, :] + 1.0)
    zpad = jnp.zeros((N1 - N, D), jnp.float32)
    y_ref[0:N, :] = xw0 * n0
    y_ref[N:N1, :] = zpad
    y_ref[N1:N1 + N, :] = xw1 * n1
    y_ref[N1 + N:, :] = zpad


def _tc_scale(h, W0, W1, deg):
    return pl.pallas_call(
        _tc_scale_body,
        out_shape=jax.ShapeDtypeStruct((2 * N1, D), jnp.float32),
    )(h, W0, W1, deg)


BR = 1280
NB = N1 // BR


def _tc_tail_a_body(acc0_ref, acc1_ref, deg0_ref, deg1_ref,
                    b0_ref, a0_ref, b1_ref, a1_ref,
                    wfc_ref, bfc_ref, e0_ref, e1_ref, s_ref):
    i = pl.program_id(0)
    n0 = lax.rsqrt(deg0_ref[0] + 1.0)
    n1 = lax.rsqrt(deg1_ref[0] + 1.0)
    e0 = acc0_ref[0] * n0 + b0_ref[...]
    e1 = acc1_ref[0] * n1 + b1_ref[...]
    e0 = jnp.where(e0 > 0, e0, e0 * a0_ref[...])
    e1 = jnp.where(e1 > 0, e1, e1 * a1_ref[...])
    e0_ref[...] = e0
    e1_ref[...] = e1

    dn = (((1,), (0,)), ((), ()))
    wfc = wfc_ref[...]
    bfc = bfc_ref[...]
    rows = lax.broadcasted_iota(jnp.int32, (BR, 1), 0) + i * BR
    valid = (rows < N).astype(jnp.float32)
    t0 = jnp.tanh(lax.dot_general(e0, wfc, dn,
                                  precision=lax.Precision.HIGHEST,
                                  preferred_element_type=jnp.float32) + bfc)
    t1 = jnp.tanh(lax.dot_general(e1, wfc, dn,
                                  precision=lax.Precision.HIGHEST,
                                  preferred_element_type=jnp.float32) + bfc)
    p0 = jnp.sum(t0 * valid, axis=0, keepdims=True)
    p1 = jnp.sum(t1 * valid, axis=0, keepdims=True)
    part = jnp.concatenate([p0, p1], axis=0)

    @pl.when(i == 0)
    def _():
        s_ref[...] = part

    @pl.when(i > 0)
    def _():
        s_ref[...] = s_ref[...] + part


def _tc_tail_b_body(e0_ref, e1_ref, s_ref, att_ref, z_ref):
    att = att_ref[...]
    w0 = jnp.sum(s_ref[0:1, :] * att) / N
    w1 = jnp.sum(s_ref[1:2, :] * att) / N
    m = jnp.maximum(w0, w1)
    p0 = jnp.exp(w0 - m)
    p1 = jnp.exp(w1 - m)
    inv = 1.0 / (p0 + p1)
    z_ref[...] = (p0 * inv) * e0_ref[0:N, :] + (p1 * inv) * e1_ref[0:N, :]


def _tc_tail(acc, deg, b0, alpha0, b1, alpha1, Wfc, bfc, att):
    acc3 = acc.reshape(2, N1, D)
    deg3 = deg.reshape(2, N1, D)
    e0, e1, s = pl.pallas_call(
        _tc_tail_a_body,
        grid=(NB,),
        in_specs=[
            pl.BlockSpec((1, BR, D), lambda i: (0, i, 0)),
            pl.BlockSpec((1, BR, D), lambda i: (1, i, 0)),
            pl.BlockSpec((1, BR, D), lambda i: (0, i, 0)),
            pl.BlockSpec((1, BR, D), lambda i: (1, i, 0)),
            pl.BlockSpec((1, D), lambda i: (0, 0)),
            pl.BlockSpec((1, 1), lambda i: (0, 0)),
            pl.BlockSpec((1, D), lambda i: (0, 0)),
            pl.BlockSpec((1, 1), lambda i: (0, 0)),
            pl.BlockSpec((D, D), lambda i: (0, 0)),
            pl.BlockSpec((1, D), lambda i: (0, 0)),
        ],
        out_specs=[
            pl.BlockSpec((BR, D), lambda i: (i, 0)),
            pl.BlockSpec((BR, D), lambda i: (i, 0)),
            pl.BlockSpec((2, D), lambda i: (0, 0)),
        ],
        out_shape=[jax.ShapeDtypeStruct((N1, D), jnp.float32),
                   jax.ShapeDtypeStruct((N1, D), jnp.float32),
                   jax.ShapeDtypeStruct((2, D), jnp.float32)],
    )(acc3, acc3, deg3, deg3, b0, alpha0, b1, alpha1, Wfc, bfc)
    return pl.pallas_call(
        _tc_tail_b_body,
        out_shape=jax.ShapeDtypeStruct((N, D), jnp.float32),
    )(e0, e1, s, att)


def kernel(h, edge_index_0, edge_index_1, W0, b0, alpha0, W1, b1, alpha1,
           Wfc, bfc, att):
    src0, dst0 = edge_index_0[0], edge_index_0[1]
    src1, dst1 = edge_index_1[0], edge_index_1[1]

    pad = (N + (jnp.arange(EPAD, dtype=jnp.int32) % (N1 - N)))
    src_h = jnp.stack([
        jnp.concatenate([src0, pad]).reshape(R, CHUNK),
        jnp.concatenate([src1 + N1, pad + N1]).reshape(R, CHUNK),
    ])
    dst_h = jnp.stack([
        jnp.concatenate([dst0, pad]).reshape(R, CHUNK),
        jnp.concatenate([dst1, pad]).reshape(R, CHUNK),
    ])

    deg_vec = _sc_degree(dst_h)
    deg = jnp.broadcast_to(deg_vec[:, None], (2 * N1, D))
    y_flat = _tc_scale(h, W0, W1, deg)
    acc = _sc_scatter(src_h, dst_h, y_flat)
    z = _tc_tail(acc, deg,
                 b0.reshape(1, D), alpha0.reshape(1, 1),
                 b1.reshape(1, D), alpha1.reshape(1, 1),
                 Wfc, bfc.reshape(1, D), att.reshape(1, D))
    return z


# speedup vs baseline: 1.2154x; 1.2154x over previous
import dataclasses
import functools

import jax
import jax.numpy as jnp
from jax import lax
from jax.experimental import pallas as pl
from jax.experimental.pallas import tpu as pltpu
from jax.experimental.pallas import tpu_sc as plsc

N = 10000
D = 128
E = 320000

NSUB = 16
N1 = 10240
NS = N1 // NSUB
CHUNK = 128
R = 2560
RPS = R // NSUB
EPAD = R * CHUNK - E
G = 32
NG = RPS // G


def _sc_degree_body(dst_hbm, deg_hbm, idx_v, deg_local, red_v, cout_v, deg_sh):
    cid = lax.axis_index("c")
    sid = lax.axis_index("s")

    pltpu.sync_copy(dst_hbm.at[cid, pl.ds(sid * RPS, RPS)], idx_v)

    @pl.loop(0, N1 // 16)
    def _(i):
        deg_local[pl.ds(i * 16, 16)] = jnp.zeros((16,), jnp.float32)

    ones16 = jnp.ones((16,), jnp.float32)

    @pl.loop(0, RPS)
    def _(i):
        @pl.loop(0, CHUNK // 16)
        def _(j):
            idx16 = idx_v[i, pl.ds(j * 16, 16)]
            plsc.addupdate_scatter(deg_local, [idx16], ones16)

    pltpu.sync_copy(deg_local, deg_sh.at[sid])
    plsc.subcore_barrier()
    pltpu.sync_copy(deg_sh.at[:, pl.ds(sid * NS, NS)], red_v)

    @pl.loop(0, NS // 16)
    def _(k):
        acc = red_v[0, pl.ds(k * 16, 16)]
        for rsub in range(1, NSUB):
            acc = acc + red_v[rsub, pl.ds(k * 16, 16)]
        cout_v[pl.ds(k * 16, 16)] = acc

    pltpu.sync_copy(cout_v, deg_hbm.at[pl.ds(cid * N1 + sid * NS, NS)])


def _sc_degree(dst_h):
    mesh = plsc.VectorSubcoreMesh(core_axis_name="c", subcore_axis_name="s")
    cp = pltpu.CompilerParams()
    if "needs_layout_passes" in pltpu.CompilerParams.__dataclass_fields__:
        cp = dataclasses.replace(cp, needs_layout_passes=False)
    fn = pl.kernel(
        _sc_degree_body,
        out_type=jax.ShapeDtypeStruct((2 * N1,), jnp.float32),
        mesh=mesh,
        compiler_params=cp,
        scratch_types=[
            pltpu.VMEM((RPS, CHUNK), jnp.int32),
            pltpu.VMEM((N1,), jnp.float32),
            pltpu.VMEM((NSUB, NS), jnp.float32),
            pltpu.VMEM((NS,), jnp.float32),
            pltpu.VMEM_SHARED((NSUB, N1), jnp.float32),
        ],
    )
    return fn(dst_h)


def _sc_scatter_body(src_hbm, dst_hbm, y_hbm, acc_hbm,
                     src_st, dst_st, rows_a, rows_b,
                     acc_sh, sem_a, sem_b):
    cid = lax.axis_index("c")
    sid = lax.axis_index("s")

    pltpu.sync_copy(y_hbm.at[pl.ds(cid * N1 + sid * NS, NS)],
                    acc_sh.at[pl.ds(sid * NS, NS)])
    plsc.subcore_barrier()

    def issue(r, rows_v, sem):
        pltpu.async_copy(y_hbm.at[src_st.at[r]], rows_v, sem)

    def drain(r, rows_v, sem):
        pltpu.make_async_copy(y_hbm.at[src_st.at[r]], rows_v, sem).wait()
        pltpu.sync_copy(rows_v, acc_sh.at[dst_st.at[r]], add=True)

    @pl.loop(0, NG)
    def _(g):
        base = sid * RPS + g * G
        pltpu.sync_copy(src_hbm.at[cid, pl.ds(base, G)], src_st)
        pltpu.sync_copy(dst_hbm.at[cid, pl.ds(base, G)], dst_st)

        issue(0, rows_a, sem_a)

        @pl.loop(0, G // 2 - 1)
        def _(i):
            issue(2 * i + 1, rows_b, sem_b)
            drain(2 * i, rows_a, sem_a)
            issue(2 * i + 2, rows_a, sem_a)
            drain(2 * i + 1, rows_b, sem_b)

        issue(G - 1, rows_b, sem_b)
        drain(G - 2, rows_a, sem_a)
        drain(G - 1, rows_b, sem_b)

    plsc.subcore_barrier()
    pltpu.sync_copy(acc_sh.at[pl.ds(sid * NS, NS)],
                    acc_hbm.at[pl.ds(cid * N1 + sid * NS, NS)])


def _sc_scatter(src_h, dst_h, y_flat):
    mesh = plsc.VectorSubcoreMesh(core_axis_name="c", subcore_axis_name="s")
    fn = pl.kernel(
        _sc_scatter_body,
        out_type=jax.ShapeDtypeStruct((2 * N1, D), jnp.float32),
        mesh=mesh,
        scratch_types=[
            pltpu.VMEM((G, CHUNK), jnp.int32),
            pltpu.VMEM((G, CHUNK), jnp.int32),
            pltpu.VMEM((CHUNK, D), jnp.float32),
            pltpu.VMEM((CHUNK, D), jnp.float32),
            pltpu.VMEM_SHARED((N1, D), jnp.float32),
            pltpu.SemaphoreType.DMA,
            pltpu.SemaphoreType.DMA,
        ],
    )
    return fn(src_h, dst_h, y_flat)


def _tc_scale_body(h_ref, w0_ref, w1_ref, deg_ref, y_ref):
    dn = (((1,), (0,)), ((), ()))
    h = h_ref[...]
    xw0 = lax.dot_general(h, w0_ref[...], dn,
                          precision=lax.Precision.HIGHEST,
                          preferred_element_type=jnp.float32)
    xw1 = lax.dot_general(h, w1_ref[...], dn,
                          precision=lax.Precision.HIGHEST,
                          preferred_element_type=jnp.float32)
    n0 = lax.rsqrt(deg_ref[0:N, :] + 1.0)
    n1 = lax.rsqrt(deg_ref[N1:N1 + N, :] + 1.0)
    zpad = jnp.zeros((N1 - N, D), jnp.float32)
    y_ref[0:N, :] = xw0 * n0
    y_ref[N:N1, :] = zpad
    y_ref[N1:N1 + N, :] = xw1 * n1
    y_ref[N1 + N:, :] = zpad


def _tc_scale(h, W0, W1, deg):
    return pl.pallas_call(
        _tc_scale_body,
        out_shape=jax.ShapeDtypeStruct((2 * N1, D), jnp.float32),
    )(h, W0, W1, deg)


BR = 1280
NB = N1 // BR


def _tc_tail_a_body(acc0_ref, acc1_ref, deg0_ref, deg1_ref,
                    b0_ref, a0_ref, b1_ref, a1_ref,
                    wfc_ref, bfc_ref, e0_ref, e1_ref, s_ref):
    i = pl.program_id(0)
    n0 = lax.rsqrt(deg0_ref[0] + 1.0)
    n1 = lax.rsqrt(deg1_ref[0] + 1.0)
    e0 = acc0_ref[0] * n0 + b0_ref[...]
    e1 = acc1_ref[0] * n1 + b1_ref[...]
    e0 = jnp.where(e0 > 0, e0, e0 * a0_ref[...])
    e1 = jnp.where(e1 > 0, e1, e1 * a1_ref[...])
    e0_ref[...] = e0
    e1_ref[...] = e1

    dn = (((1,), (0,)), ((), ()))
    wfc = wfc_ref[...]
    bfc = bfc_ref[...]
    rows = lax.broadcasted_iota(jnp.int32, (BR, 1), 0) + i * BR
    valid = (rows < N).astype(jnp.float32)
    t0 = jnp.tanh(lax.dot_general(e0, wfc, dn,
                                  precision=lax.Precision.HIGHEST,
                                  preferred_element_type=jnp.float32) + bfc)
    t1 = jnp.tanh(lax.dot_general(e1, wfc, dn,
                                  precision=lax.Precision.HIGHEST,
                                  preferred_element_type=jnp.float32) + bfc)
    p0 = jnp.sum(t0 * valid, axis=0, keepdims=True)
    p1 = jnp.sum(t1 * valid, axis=0, keepdims=True)
    part = jnp.concatenate([p0, p1], axis=0)

    @pl.when(i == 0)
    def _():
        s_ref[...] = part

    @pl.when(i > 0)
    def _():
        s_ref[...] = s_ref[...] + part


def _tc_tail_b_body(e0_ref, e1_ref, s_ref, att_ref, z_ref):
    att = att_ref[...]
    w0 = jnp.sum(s_ref[0:1, :] * att) / N
    w1 = jnp.sum(s_ref[1:2, :] * att) / N
    m = jnp.maximum(w0, w1)
    p0 = jnp.exp(w0 - m)
    p1 = jnp.exp(w1 - m)
    inv = 1.0 / (p0 + p1)
    z_ref[...] = (p0 * inv) * e0_ref[0:N, :] + (p1 * inv) * e1_ref[0:N, :]


def _tc_tail(acc, deg, b0, alpha0, b1, alpha1, Wfc, bfc, att):
    acc3 = acc.reshape(2, N1, D)
    deg3 = deg.reshape(2, N1, D)
    e0, e1, s = pl.pallas_call(
        _tc_tail_a_body,
        grid=(NB,),
        in_specs=[
            pl.BlockSpec((1, BR, D), lambda i: (0, i, 0)),
            pl.BlockSpec((1, BR, D), lambda i: (1, i, 0)),
            pl.BlockSpec((1, BR, D), lambda i: (0, i, 0)),
            pl.BlockSpec((1, BR, D), lambda i: (1, i, 0)),
            pl.BlockSpec((1, D), lambda i: (0, 0)),
            pl.BlockSpec((1, 1), lambda i: (0, 0)),
            pl.BlockSpec((1, D), lambda i: (0, 0)),
            pl.BlockSpec((1, 1), lambda i: (0, 0)),
            pl.BlockSpec((D, D), lambda i: (0, 0)),
            pl.BlockSpec((1, D), lambda i: (0, 0)),
        ],
        out_specs=[
            pl.BlockSpec((BR, D), lambda i: (i, 0)),
            pl.BlockSpec((BR, D), lambda i: (i, 0)),
            pl.BlockSpec((2, D), lambda i: (0, 0)),
        ],
        out_shape=[jax.ShapeDtypeStruct((N1, D), jnp.float32),
                   jax.ShapeDtypeStruct((N1, D), jnp.float32),
                   jax.ShapeDtypeStruct((2, D), jnp.float32)],
    )(acc3, acc3, deg3, deg3, b0, alpha0, b1, alpha1, Wfc, bfc)
    return pl.pallas_call(
        _tc_tail_b_body,
        out_shape=jax.ShapeDtypeStruct((N, D), jnp.float32),
    )(e0, e1, s, att)


def kernel(h, edge_index_0, edge_index_1, W0, b0, alpha0, W1, b1, alpha1,
           Wfc, bfc, att):
    src0, dst0 = edge_index_0[0], edge_index_0[1]
    src1, dst1 = edge_index_1[0], edge_index_1[1]

    pad = (N + (jnp.arange(EPAD, dtype=jnp.int32) % (N1 - N)))
    src_h = jnp.stack([
        jnp.concatenate([src0, pad]).reshape(R, CHUNK),
        jnp.concatenate([src1 + N1, pad + N1]).reshape(R, CHUNK),
    ])
    dst_h = jnp.stack([
        jnp.concatenate([dst0, pad]).reshape(R, CHUNK),
        jnp.concatenate([dst1, pad]).reshape(R, CHUNK),
    ])

    deg_vec = _sc_degree(dst_h)
    deg = jnp.broadcast_to(deg_vec[:, None], (2 * N1, D))
    y_flat = _tc_scale(h, W0, W1, deg)
    acc = _sc_scatter(src_h, dst_h, y_flat)
    z = _tc_tail(acc, deg,
                 b0.reshape(1, D), alpha0.reshape(1, 1),
                 b1.reshape(1, D), alpha1.reshape(1, 1),
                 Wfc, bfc.reshape(1, D), att.reshape(1, D))
    return z


# speedup vs baseline: 1.3170x; 1.0836x over previous
import dataclasses
import functools

import jax
import jax.numpy as jnp
from jax import lax
from jax.experimental import pallas as pl
from jax.experimental.pallas import tpu as pltpu
from jax.experimental.pallas import tpu_sc as plsc

N = 10000
D = 128
E = 320000

NSUB = 16
N1 = 10240
NS = N1 // NSUB
CHUNK = 128
R = 2560
RPS = R // NSUB
EPAD = R * CHUNK - E
G = 32
NG = RPS // G


def _sc_degree_body(dst_hbm, deg_hbm, idx_v, deg_local, red_v, cout_v, rowb_v, deg_sh):
    cid = lax.axis_index("c")
    sid = lax.axis_index("s")

    pltpu.sync_copy(dst_hbm.at[cid, pl.ds(sid * RPS, RPS)], idx_v)

    @pl.loop(0, N1 // 16)
    def _(i):
        deg_local[pl.ds(i * 16, 16)] = jnp.zeros((16,), jnp.float32)

    ones16 = jnp.ones((16,), jnp.float32)

    @pl.loop(0, RPS)
    def _(i):
        @pl.loop(0, CHUNK // 16)
        def _(j):
            idx16 = idx_v[i, pl.ds(j * 16, 16)]
            plsc.addupdate_scatter(deg_local, [idx16], ones16)

    pltpu.sync_copy(deg_local, deg_sh.at[sid])
    plsc.subcore_barrier()
    pltpu.sync_copy(deg_sh.at[:, pl.ds(sid * NS, NS)], red_v)

    @pl.loop(0, NS // 16)
    def _(k):
        acc = red_v[0, pl.ds(k * 16, 16)]
        for rsub in range(1, NSUB):
            acc = acc + red_v[rsub, pl.ds(k * 16, 16)]
        cout_v[pl.ds(k * 16, 16)] = acc

    zero16 = jnp.zeros((16,), jnp.int32)

    @pl.loop(0, NS // CHUNK)
    def _(b):
        @pl.loop(0, CHUNK)
        def _(j):
            val16 = plsc.load_gather(cout_v, [zero16 + (b * CHUNK + j)])
            for half in range(D // 16):
                rowb_v[j, pl.ds(half * 16, 16)] = val16

        pltpu.sync_copy(
            rowb_v,
            deg_hbm.at[pl.ds(cid * N1 + sid * NS + b * CHUNK, CHUNK)])


def _sc_degree(dst_h):
    mesh = plsc.VectorSubcoreMesh(core_axis_name="c", subcore_axis_name="s")
    cp = pltpu.CompilerParams()
    if "needs_layout_passes" in pltpu.CompilerParams.__dataclass_fields__:
        cp = dataclasses.replace(cp, needs_layout_passes=False)
    fn = pl.kernel(
        _sc_degree_body,
        out_type=jax.ShapeDtypeStruct((2 * N1, D), jnp.float32),
        mesh=mesh,
        compiler_params=cp,
        scratch_types=[
            pltpu.VMEM((RPS, CHUNK), jnp.int32),
            pltpu.VMEM((N1,), jnp.float32),
            pltpu.VMEM((NSUB, NS), jnp.float32),
            pltpu.VMEM((NS,), jnp.float32),
            pltpu.VMEM((CHUNK, D), jnp.float32),
            pltpu.VMEM_SHARED((NSUB, N1), jnp.float32),
        ],
    )
    return fn(dst_h)


def _sc_scatter_body(src_hbm, dst_hbm, y_hbm, acc_hbm,
                     src_st, dst_st, rows_a, rows_b,
                     acc_sh, sem_a, sem_b):
    cid = lax.axis_index("c")
    sid = lax.axis_index("s")

    pltpu.sync_copy(y_hbm.at[pl.ds(cid * N1 + sid * NS, NS)],
                    acc_sh.at[pl.ds(sid * NS, NS)])
    plsc.subcore_barrier()

    def issue(r, rows_v, sem):
        pltpu.async_copy(y_hbm.at[src_st.at[r]], rows_v, sem)

    def drain(r, rows_v, sem):
        pltpu.make_async_copy(y_hbm.at[src_st.at[r]], rows_v, sem).wait()
        pltpu.sync_copy(rows_v, acc_sh.at[dst_st.at[r]], add=True)

    @pl.loop(0, NG)
    def _(g):
        base = sid * RPS + g * G
        pltpu.sync_copy(src_hbm.at[cid, pl.ds(base, G)], src_st)
        pltpu.sync_copy(dst_hbm.at[cid, pl.ds(base, G)], dst_st)

        issue(0, rows_a, sem_a)

        @pl.loop(0, G // 2 - 1)
        def _(i):
            issue(2 * i + 1, rows_b, sem_b)
            drain(2 * i, rows_a, sem_a)
            issue(2 * i + 2, rows_a, sem_a)
            drain(2 * i + 1, rows_b, sem_b)

        issue(G - 1, rows_b, sem_b)
        drain(G - 2, rows_a, sem_a)
        drain(G - 1, rows_b, sem_b)

    plsc.subcore_barrier()
    pltpu.sync_copy(acc_sh.at[pl.ds(sid * NS, NS)],
                    acc_hbm.at[pl.ds(cid * N1 + sid * NS, NS)])


def _sc_scatter(src_h, dst_h, y_flat):
    mesh = plsc.VectorSubcoreMesh(core_axis_name="c", subcore_axis_name="s")
    fn = pl.kernel(
        _sc_scatter_body,
        out_type=jax.ShapeDtypeStruct((2 * N1, D), jnp.float32),
        mesh=mesh,
        scratch_types=[
            pltpu.VMEM((G, CHUNK), jnp.int32),
            pltpu.VMEM((G, CHUNK), jnp.int32),
            pltpu.VMEM((CHUNK, D), jnp.float32),
            pltpu.VMEM((CHUNK, D), jnp.float32),
            pltpu.VMEM_SHARED((N1, D), jnp.float32),
            pltpu.SemaphoreType.DMA,
            pltpu.SemaphoreType.DMA,
        ],
    )
    return fn(src_h, dst_h, y_flat)


EDGE_ROWS = E // CHUNK
PAD_ROWS = R - EDGE_ROWS


def _tc_pack_body(ei0_ref, ei1_ref, src_ref, dst_ref):
    i0 = lax.broadcasted_iota(jnp.int32, (PAD_ROWS, CHUNK), 0)
    i1 = lax.broadcasted_iota(jnp.int32, (PAD_ROWS, CHUNK), 1)
    padv = N + lax.rem(i0 * CHUNK + i1, N1 - N)
    src_ref[0, 0:EDGE_ROWS, :] = ei0_ref[0]
    src_ref[0, EDGE_ROWS:, :] = padv
    src_ref[1, 0:EDGE_ROWS, :] = ei1_ref[0] + N1
    src_ref[1, EDGE_ROWS:, :] = padv + N1
    dst_ref[0, 0:EDGE_ROWS, :] = ei0_ref[1]
    dst_ref[0, EDGE_ROWS:, :] = padv
    dst_ref[1, 0:EDGE_ROWS, :] = ei1_ref[1]
    dst_ref[1, EDGE_ROWS:, :] = padv


def _tc_pack(ei0, ei1):
    return pl.pallas_call(
        _tc_pack_body,
        out_shape=[jax.ShapeDtypeStruct((2, R, CHUNK), jnp.int32)] * 2,
    )(ei0, ei1)


def _tc_scale_body(h_ref, w0_ref, w1_ref, deg_ref, y_ref):
    dn = (((1,), (0,)), ((), ()))
    h = h_ref[...]
    xw0 = lax.dot_general(h, w0_ref[...], dn,
                          precision=lax.Precision.HIGHEST,
                          preferred_element_type=jnp.float32)
    xw1 = lax.dot_general(h, w1_ref[...], dn,
                          precision=lax.Precision.HIGHEST,
                          preferred_element_type=jnp.float32)
    n0 = lax.rsqrt(deg_ref[0:N, 0:1] + 1.0)
    n1 = lax.rsqrt(deg_ref[N1:N1 + N, 0:1] + 1.0)
    zpad = jnp.zeros((N1 - N, D), jnp.float32)
    y_ref[0:N, :] = xw0 * n0
    y_ref[N:N1, :] = zpad
    y_ref[N1:N1 + N, :] = xw1 * n1
    y_ref[N1 + N:, :] = zpad


def _tc_scale(h, W0, W1, deg):
    return pl.pallas_call(
        _tc_scale_body,
        out_shape=jax.ShapeDtypeStruct((2 * N1, D), jnp.float32),
    )(h, W0, W1, deg)


BR = 1280
NB = N1 // BR


def _tc_tail_a_body(acc0_ref, acc1_ref, deg0_ref, deg1_ref,
                    b0_ref, a0_ref, b1_ref, a1_ref,
                    wfc_ref, bfc_ref, e0_ref, e1_ref, s_ref):
    i = pl.program_id(0)
    n0 = lax.rsqrt(deg0_ref[0, :, 0:1] + 1.0)
    n1 = lax.rsqrt(deg1_ref[0, :, 0:1] + 1.0)
    e0 = acc0_ref[0] * n0 + b0_ref[...]
    e1 = acc1_ref[0] * n1 + b1_ref[...]
    e0 = jnp.where(e0 > 0, e0, e0 * a0_ref[...])
    e1 = jnp.where(e1 > 0, e1, e1 * a1_ref[...])
    e0_ref[...] = e0
    e1_ref[...] = e1

    dn = (((1,), (0,)), ((), ()))
    wfc = wfc_ref[...]
    bfc = bfc_ref[...]
    rows = lax.broadcasted_iota(jnp.int32, (BR, 1), 0) + i * BR
    valid = (rows < N).astype(jnp.float32)
    t0 = jnp.tanh(lax.dot_general(e0, wfc, dn,
                                  precision=lax.Precision.HIGHEST,
                                  preferred_element_type=jnp.float32) + bfc)
    t1 = jnp.tanh(lax.dot_general(e1, wfc, dn,
                                  precision=lax.Precision.HIGHEST,
                                  preferred_element_type=jnp.float32) + bfc)
    p0 = jnp.sum(t0 * valid, axis=0, keepdims=True)
    p1 = jnp.sum(t1 * valid, axis=0, keepdims=True)
    part = jnp.concatenate([p0, p1], axis=0)

    @pl.when(i == 0)
    def _():
        s_ref[...] = part

    @pl.when(i > 0)
    def _():
        s_ref[...] = s_ref[...] + part


def _tc_tail_b_body(e0_ref, e1_ref, s_ref, att_ref, z_ref):
    att = att_ref[...]
    w0 = jnp.sum(s_ref[0:1, :] * att) / N
    w1 = jnp.sum(s_ref[1:2, :] * att) / N
    m = jnp.maximum(w0, w1)
    p0 = jnp.exp(w0 - m)
    p1 = jnp.exp(w1 - m)
    inv = 1.0 / (p0 + p1)
    z_ref[...] = (p0 * inv) * e0_ref[0:N, :] + (p1 * inv) * e1_ref[0:N, :]


def _tc_tail(acc, deg, b0, alpha0, b1, alpha1, Wfc, bfc, att):
    acc3 = acc.reshape(2, N1, D)
    deg3 = deg.reshape(2, N1, D)
    e0, e1, s = pl.pallas_call(
        _tc_tail_a_body,
        grid=(NB,),
        in_specs=[
            pl.BlockSpec((1, BR, D), lambda i: (0, i, 0)),
            pl.BlockSpec((1, BR, D), lambda i: (1, i, 0)),
            pl.BlockSpec((1, BR, D), lambda i: (0, i, 0)),
            pl.BlockSpec((1, BR, D), lambda i: (1, i, 0)),
            pl.BlockSpec((1, D), lambda i: (0, 0)),
            pl.BlockSpec((1, 1), lambda i: (0, 0)),
            pl.BlockSpec((1, D), lambda i: (0, 0)),
            pl.BlockSpec((1, 1), lambda i: (0, 0)),
            pl.BlockSpec((D, D), lambda i: (0, 0)),
            pl.BlockSpec((1, D), lambda i: (0, 0)),
        ],
        out_specs=[
            pl.BlockSpec((BR, D), lambda i: (i, 0)),
            pl.BlockSpec((BR, D), lambda i: (i, 0)),
            pl.BlockSpec((2, D), lambda i: (0, 0)),
        ],
        out_shape=[jax.ShapeDtypeStruct((N1, D), jnp.float32),
                   jax.ShapeDtypeStruct((N1, D), jnp.float32),
                   jax.ShapeDtypeStruct((2, D), jnp.float32)],
    )(acc3, acc3, deg3, deg3, b0, alpha0, b1, alpha1, Wfc, bfc)
    return pl.pallas_call(
        _tc_tail_b_body,
        out_shape=jax.ShapeDtypeStruct((N, D), jnp.float32),
    )(e0, e1, s, att)


def kernel(h, edge_index_0, edge_index_1, W0, b0, alpha0, W1, b1, alpha1,
           Wfc, bfc, att):
    ei0 = edge_index_0.reshape(2, EDGE_ROWS, CHUNK)
    ei1 = edge_index_1.reshape(2, EDGE_ROWS, CHUNK)
    src_h, dst_h = _tc_pack(ei0, ei1)

    deg = _sc_degree(dst_h)
    y_flat = _tc_scale(h, W0, W1, deg)
    acc = _sc_scatter(src_h, dst_h, y_flat)
    z = _tc_tail(acc, deg,
                 b0.reshape(1, D), alpha0.reshape(1, 1),
                 b1.reshape(1, D), alpha1.reshape(1, 1),
                 Wfc, bfc.reshape(1, D), att.reshape(1, D))
    return z
